# Initial kernel scaffold; baseline (speedup 1.0000x reference)
import functools

import jax
import jax.numpy as jnp
from jax import lax
from jax.experimental import pallas as pl
from jax.experimental.pallas import tpu as pltpu
from jax.experimental.pallas import tpu_sc as plsc

N = 10000
E = 320000
DF = 128
DE = 16
D = 64
G = 64

NCORES = 2
NSUB = 16
NW = NCORES * NSUB
EW = E // NW
CH = 80
NCHUNK = EW // CH
STRIPE = 640
NP = NSUB * STRIPE

F32 = jnp.float32

_L1P = (0.99999997, -0.49999502, 0.33319278, -0.24844407, 0.19111539,
        -0.13674945, 0.07836325, -0.02958924, 0.00525359)


def _sigmoid(x):
    e = jnp.exp(-jnp.abs(x))
    r = 1.0 / (1.0 + e)
    return jnp.where(x >= 0, r, 1.0 - r)


def _softplus(x):
    t = jnp.exp(-jnp.abs(x))
    p = jnp.full(x.shape, _L1P[8], F32)
    for k in range(7, -1, -1):
        p = p * t + _L1P[k]
    return jnp.maximum(x, 0.0) + t * p


def _pre_body(x_ref, w_ref, b_ref, o_ref):
    o_ref[...] = jax.nn.relu(
        jnp.dot(x_ref[...], w_ref[...], preferred_element_type=F32)
        + b_ref[...])


def _pre(x, w, b):
    blk = 1000
    return pl.pallas_call(
        _pre_body,
        grid=(N // blk,),
        in_specs=[
            pl.BlockSpec((blk, DF), lambda i: (i, 0)),
            pl.BlockSpec((DF, D), lambda i: (0, 0)),
            pl.BlockSpec((1, D), lambda i: (0, 0)),
        ],
        out_specs=pl.BlockSpec((blk, D), lambda i: (i, 0)),
        out_shape=jax.ShapeDtypeStruct((N, D), F32),
    )(x, w, b.reshape(1, D))


def _edgeconst_body(ea_ref, w_ref, b_ref, o0_ref, o1_ref, o2_ref):
    ea = ea_ref[...]
    for l, o_ref in enumerate((o0_ref, o1_ref, o2_ref)):
        o_ref[...] = (
            jnp.dot(ea, w_ref[l], preferred_element_type=F32) + b_ref[l])


def _edgeconst(ea, wcat, bcat):
    blk = 4000
    return pl.pallas_call(
        _edgeconst_body,
        grid=(E // blk,),
        in_specs=[
            pl.BlockSpec((blk, DE), lambda i: (i, 0)),
            pl.BlockSpec((3, DE, 2 * D), lambda i: (0, 0, 0)),
            pl.BlockSpec((3, 1, 2 * D), lambda i: (0, 0, 0)),
        ],
        out_specs=[pl.BlockSpec((blk, 2 * D), lambda i: (i, 0))] * 3,
        out_shape=[jax.ShapeDtypeStruct((E, 2 * D), F32)] * 3,
    )(ea, wcat, bcat)


def _tables_body(h_ref, wd_ref, ws_ref, td_ref, ts_ref):
    h = h_ref[...]
    td_ref[...] = jnp.dot(h, wd_ref[...], preferred_element_type=F32)
    ts_ref[...] = jnp.dot(h, ws_ref[...], preferred_element_type=F32)


def _tables(h, wd, ws):
    blk = 1000
    return pl.pallas_call(
        _tables_body,
        grid=(N // blk,),
        in_specs=[
            pl.BlockSpec((blk, D), lambda i: (i, 0)),
            pl.BlockSpec((D, 2 * D), lambda i: (0, 0)),
            pl.BlockSpec((D, 2 * D), lambda i: (0, 0)),
        ],
        out_specs=[pl.BlockSpec((blk, 2 * D), lambda i: (i, 0))] * 2,
        out_shape=[jax.ShapeDtypeStruct((N, 2 * D), F32)] * 2,
    )(h, wd, ws)


def _update_bn(h_ref, a0_ref, a1_ref, c0_ref, c1_ref, g_ref, be_ref):
    cnt = jnp.maximum(c0_ref[0] + c1_ref[0], 1.0)
    hu = h_ref[...] + (a0_ref[0] + a1_ref[0]) / cnt
    mu = jnp.mean(hu, axis=0, keepdims=True)
    var = jnp.mean((hu - mu) ** 2, axis=0, keepdims=True)
    return (hu - mu) * lax.rsqrt(var + 1e-5) * g_ref[...] + be_ref[...]


def _bn_tables_body(h_ref, a0_ref, a1_ref, c0_ref, c1_ref, g_ref, be_ref,
                    wd_ref, ws_ref, hn_ref, td_ref, ts_ref):
    hn = _update_bn(h_ref, a0_ref, a1_ref, c0_ref, c1_ref, g_ref, be_ref)
    hn_ref[...] = hn
    td_ref[...] = jnp.dot(hn, wd_ref[...], preferred_element_type=F32)
    ts_ref[...] = jnp.dot(hn, ws_ref[...], preferred_element_type=F32)


def _bn_tables(h, acc, cnt, g, be, wd, ws):
    accspec = [
        pl.BlockSpec((1, N, D), lambda: (0, 0, 0)),
        pl.BlockSpec((1, N, D), lambda: (1, 0, 0)),
    ]
    return pl.pallas_call(
        _bn_tables_body,
        in_specs=[pl.BlockSpec((N, D), lambda: (0, 0))]
        + accspec + accspec
        + [
            pl.BlockSpec((1, D), lambda: (0, 0)),
            pl.BlockSpec((1, D), lambda: (0, 0)),
            pl.BlockSpec((D, 2 * D), lambda: (0, 0)),
            pl.BlockSpec((D, 2 * D), lambda: (0, 0)),
        ],
        out_specs=[
            pl.BlockSpec((N, D), lambda: (0, 0)),
            pl.BlockSpec((N, 2 * D), lambda: (0, 0)),
            pl.BlockSpec((N, 2 * D), lambda: (0, 0)),
        ],
        out_shape=[
            jax.ShapeDtypeStruct((N, D), F32),
            jax.ShapeDtypeStruct((N, 2 * D), F32),
            jax.ShapeDtypeStruct((N, 2 * D), F32),
        ],
    )(h, acc, acc, cnt, cnt, g.reshape(1, D), be.reshape(1, D), wd, ws)


def _final_body(h_ref, a0_ref, a1_ref, c0_ref, c1_ref, g_ref, be_ref,
                batch_ref, wp_ref, bp_ref, wo_ref, bo_ref, o_ref):
    hn = _update_bn(h_ref, a0_ref, a1_ref, c0_ref, c1_ref, g_ref, be_ref)
    gid = lax.broadcasted_iota(jnp.int32, (G, 1), 0)
    mask = (batch_ref[...] == gid).astype(F32)
    sums = jnp.dot(mask, hn, preferred_element_type=F32)
    cg = jnp.sum(mask, axis=1, keepdims=True)
    p = sums / jnp.maximum(cg, 1.0)
    p = jax.nn.relu(jnp.dot(p, wp_ref[...], preferred_element_type=F32)
                    + bp_ref[...])
    o_ref[...] = (jnp.dot(p, wo_ref[...], preferred_element_type=F32)
                  + bo_ref[...])


def _final(h, acc, cnt, g, be, batch, wp, bp, wo, bo):
    accspec = [
        pl.BlockSpec((1, N, D), lambda: (0, 0, 0)),
        pl.BlockSpec((1, N, D), lambda: (1, 0, 0)),
    ]
    return pl.pallas_call(
        _final_body,
        in_specs=[pl.BlockSpec((N, D), lambda: (0, 0))]
        + accspec + accspec
        + [
            pl.BlockSpec((1, D), lambda: (0, 0)),
            pl.BlockSpec((1, D), lambda: (0, 0)),
            pl.BlockSpec((1, N), lambda: (0, 0)),
            pl.BlockSpec((D, D), lambda: (0, 0)),
            pl.BlockSpec((1, D), lambda: (0, 0)),
            pl.BlockSpec((D, 1), lambda: (0, 0)),
            pl.BlockSpec((1, 1), lambda: (0, 0)),
        ],
        out_specs=pl.BlockSpec((G, 1), lambda: (0, 0)),
        out_shape=jax.ShapeDtypeStruct((G, 1), F32),
    )(h, acc, acc, cnt, cnt, g.reshape(1, D), be.reshape(1, D),
      batch.reshape(1, N), wp, bp.reshape(1, D), wo, bo.reshape(1, 1))


def _fill(ref, rows, value):
    def row(i, _):
        for q in range(D // 16):
            ref[i, pl.ds(q * 16, 16)] = jnp.full((16,), value, F32)
        return 0
    lax.fori_loop(0, rows, row, 0)


def _make_sc_edge(want_cnt):
    mesh = plsc.VectorSubcoreMesh(core_axis_name="c", subcore_axis_name="s")
    out_type = [jax.ShapeDtypeStruct((NCORES, NP, D), F32)]
    scratch = [
        pltpu.VMEM((CH,), jnp.int32),
        pltpu.VMEM((CH,), jnp.int32),
        pltpu.VMEM((CH, 2 * D), F32),
        pltpu.VMEM((CH, 2 * D), F32),
        pltpu.VMEM((CH, 2 * D), F32),
        pltpu.VMEM((CH, D), F32),
        pltpu.VMEM((STRIPE, D), F32),
        pltpu.VMEM_SHARED((NP, D), F32),
        pltpu.SemaphoreType.DMA,
        pltpu.SemaphoreType.DMA,
        pltpu.SemaphoreType.DMA,
    ]
    if want_cnt:
        out_type.append(jax.ShapeDtypeStruct((NCORES, NP, D), F32))
        scratch += [
            pltpu.VMEM((CH, D), F32),
            pltpu.VMEM_SHARED((NP, D), F32),
        ]

    def body(dst_hbm, src_hbm, td_hbm, ts_hbm, cfs_hbm, *rest):
        if want_cnt:
            (out_hbm, cntout_hbm, dsti, srci, gd, gs, cv, mv, zb,
             acc, sem0, sem1, sem2, ob, cacc) = rest
        else:
            (out_hbm, dsti, srci, gd, gs, cv, mv, zb,
             acc, sem0, sem1, sem2) = rest
        cid = lax.axis_index("c")
        sid = lax.axis_index("s")
        wid = cid * NSUB + sid

        _fill(zb, STRIPE, 0.0)
        pltpu.sync_copy(zb, acc.at[pl.ds(sid * STRIPE, STRIPE)])
        if want_cnt:
            pltpu.sync_copy(zb, cacc.at[pl.ds(sid * STRIPE, STRIPE)])
            _fill(ob, CH, 1.0)
        plsc.subcore_barrier()

        def chunk(c, _):
            base = wid * EW + c * CH
            pltpu.sync_copy(dst_hbm.at[pl.ds(base, CH)], dsti)
            pltpu.sync_copy(src_hbm.at[pl.ds(base, CH)], srci)
            cp0 = pltpu.async_copy(td_hbm.at[dsti], gd, sem0)
            cp1 = pltpu.async_copy(ts_hbm.at[srci], gs, sem1)
            cp2 = pltpu.async_copy(cfs_hbm.at[pl.ds(base, CH)], cv, sem2)
            cp0.wait()
            cp1.wait()
            cp2.wait()

            def row(i, _):
                for q in range(D // 16):
                    slf = pl.ds(q * 16, 16)
                    sls = pl.ds(D + q * 16, 16)
                    zf = gd[i, slf] + gs[i, slf] + cv[i, slf]
                    zs = gd[i, sls] + gs[i, sls] + cv[i, sls]
                    mv[i, slf] = _sigmoid(zf) * _softplus(zs)
                return 0
            lax.fori_loop(0, CH, row, 0)

            pltpu.sync_copy(mv, acc.at[dsti], add=True)
            if want_cnt:
                pltpu.sync_copy(ob, cacc.at[dsti], add=True)
            return 0
        lax.fori_loop(0, NCHUNK, chunk, 0)

        plsc.subcore_barrier()

        @pl.when(sid == 0)
        def _():
            pltpu.sync_copy(acc, out_hbm.at[cid])
            if want_cnt:
                pltpu.sync_copy(cacc, cntout_hbm.at[cid])

    return pl.kernel(body, out_type=out_type, mesh=mesh,
                     scratch_types=scratch)


_sc_edge_cnt = _make_sc_edge(True)
_sc_edge = _make_sc_edge(False)


def kernel(x, edge_index, edge_attr, batch,
           W_pre, b_pre,
           Wf0, bf0, Ws0, bs0, g0, be0,
           Wf1, bf1, Ws1, bs1, g1, be1,
           Wf2, bf2, Ws2, bs2, g2, be2,
           W_post, b_post, W_out, b_out):
    src = edge_index[0]
    dst = edge_index[1]
    layers = ((Wf0, bf0, Ws0, bs0, g0, be0),
              (Wf1, bf1, Ws1, bs1, g1, be1),
              (Wf2, bf2, Ws2, bs2, g2, be2))

    h = _pre(x, W_pre, b_pre)

    wcat = jnp.stack([
        jnp.concatenate([Wf[2 * D:], Ws[2 * D:]], axis=1)
        for (Wf, _, Ws, _, _, _) in layers])
    bcat = jnp.stack([
        jnp.concatenate([bf, bs]).reshape(1, 2 * D)
        for (_, bf, _, bs, _, _) in layers])
    cfs = _edgeconst(edge_attr, wcat, bcat)

    acc = cnt = None
    for l, (Wf, bf, Ws, bs, g, be) in enumerate(layers):
        wd = jnp.concatenate([Wf[:D], Ws[:D]], axis=1)
        wsrc = jnp.concatenate([Wf[D:2 * D], Ws[D:2 * D]], axis=1)
        if l == 0:
            td, ts = _tables(h, wd, wsrc)
            acc, cnt = _sc_edge_cnt(dst, src, td, ts, cfs[l])
        else:
            h, td, ts = _bn_tables(h, acc, cnt, layers[l - 1][4],
                                   layers[l - 1][5], wd, wsrc)
            (acc,) = _sc_edge(dst, src, td, ts, cfs[l])

    return _final(h, acc, cnt, g2, be2, batch,
                  W_post, b_post, W_out, b_out)


# speedup vs baseline: 1.2138x; 1.2138x over previous
import jax
import jax.numpy as jnp
from jax import lax
from jax.experimental import pallas as pl
from jax.experimental.pallas import tpu as pltpu
from jax.experimental.pallas import tpu_sc as plsc

N = 10000
E = 320000
DF = 128
DE = 16
D = 64
G = 64

NCORES = 2
NSUB = 16
NW = NCORES * NSUB
EW = E // NW
CH = 80
NCHUNK = EW // CH
STRIPE = 640
NP = NSUB * STRIPE

F32 = jnp.float32

_L1P = (0.99999997, -0.49999502, 0.33319278, -0.24844407, 0.19111539,
        -0.13674945, 0.07836325, -0.02958924, 0.00525359)


def _sigmoid(x):
    e = jnp.exp(-jnp.abs(x))
    r = 1.0 / (1.0 + e)
    return jnp.where(x >= 0, r, 1.0 - r)


def _softplus(x):
    t = jnp.exp(-jnp.abs(x))
    p = jnp.full(x.shape, _L1P[8], F32)
    for k in range(7, -1, -1):
        p = p * t + _L1P[k]
    return jnp.maximum(x, 0.0) + t * p


def _pre_body(x_ref, w_ref, b_ref, o_ref):
    o_ref[...] = jax.nn.relu(
        jnp.dot(x_ref[...], w_ref[...], preferred_element_type=F32)
        + b_ref[...])


def _pre(x, w, b):
    blk = 1000
    return pl.pallas_call(
        _pre_body,
        grid=(N // blk,),
        in_specs=[
            pl.BlockSpec((blk, DF), lambda i: (i, 0)),
            pl.BlockSpec((DF, D), lambda i: (0, 0)),
            pl.BlockSpec((1, D), lambda i: (0, 0)),
        ],
        out_specs=pl.BlockSpec((blk, D), lambda i: (i, 0)),
        out_shape=jax.ShapeDtypeStruct((N, D), F32),
    )(x, w, b.reshape(1, D))


def _edgeconst_body(ea_ref, w_ref, b_ref, o0_ref, o1_ref, o2_ref):
    ea = ea_ref[...]
    for l, o_ref in enumerate((o0_ref, o1_ref, o2_ref)):
        o_ref[...] = (
            jnp.dot(ea, w_ref[l], preferred_element_type=F32) + b_ref[l])


def _edgeconst(ea, wcat, bcat):
    blk = 4000
    return pl.pallas_call(
        _edgeconst_body,
        grid=(E // blk,),
        in_specs=[
            pl.BlockSpec((blk, DE), lambda i: (i, 0)),
            pl.BlockSpec((3, DE, 2 * D), lambda i: (0, 0, 0)),
            pl.BlockSpec((3, 1, 2 * D), lambda i: (0, 0, 0)),
        ],
        out_specs=[pl.BlockSpec((blk, 2 * D), lambda i: (i, 0))] * 3,
        out_shape=[jax.ShapeDtypeStruct((E, 2 * D), F32)] * 3,
    )(ea, wcat, bcat)


def _tables_body(h_ref, wd_ref, ws_ref, td_ref, ts_ref):
    h = h_ref[...]
    td_ref[...] = jnp.dot(h, wd_ref[...], preferred_element_type=F32)
    ts_ref[...] = jnp.dot(h, ws_ref[...], preferred_element_type=F32)


def _tables(h, wd, ws):
    blk = 1000
    return pl.pallas_call(
        _tables_body,
        grid=(N // blk,),
        in_specs=[
            pl.BlockSpec((blk, D), lambda i: (i, 0)),
            pl.BlockSpec((D, 2 * D), lambda i: (0, 0)),
            pl.BlockSpec((D, 2 * D), lambda i: (0, 0)),
        ],
        out_specs=[pl.BlockSpec((blk, 2 * D), lambda i: (i, 0))] * 2,
        out_shape=[jax.ShapeDtypeStruct((N, 2 * D), F32)] * 2,
    )(h, wd, ws)


def _update_bn(h_ref, a0_ref, a1_ref, c0_ref, c1_ref, g_ref, be_ref):
    cnt = jnp.maximum(c0_ref[0, :, :1] + c1_ref[0, :, :1], 1.0)
    hu = h_ref[...] + (a0_ref[0] + a1_ref[0]) / cnt
    mu = jnp.mean(hu, axis=0, keepdims=True)
    var = jnp.mean((hu - mu) ** 2, axis=0, keepdims=True)
    return (hu - mu) * lax.rsqrt(var + 1e-5) * g_ref[...] + be_ref[...]


def _bn_tables_body(h_ref, a0_ref, a1_ref, c0_ref, c1_ref, g_ref, be_ref,
                    wd_ref, ws_ref, hn_ref, td_ref, ts_ref):
    hn = _update_bn(h_ref, a0_ref, a1_ref, c0_ref, c1_ref, g_ref, be_ref)
    hn_ref[...] = hn
    td_ref[...] = jnp.dot(hn, wd_ref[...], preferred_element_type=F32)
    ts_ref[...] = jnp.dot(hn, ws_ref[...], preferred_element_type=F32)


_ACCSPEC = [
    pl.BlockSpec((1, N, D), lambda i: (0, 0, 0)),
    pl.BlockSpec((1, N, D), lambda i: (1, 0, 0)),
]
_CNTSPEC = [
    pl.BlockSpec((1, N, 16), lambda i: (0, 0, 0)),
    pl.BlockSpec((1, N, 16), lambda i: (1, 0, 0)),
]


def _bn_tables(h, acc, cnt, g, be, wd, ws):
    return pl.pallas_call(
        _bn_tables_body,
        grid=(1,),
        in_specs=[pl.BlockSpec((N, D), lambda i: (0, 0))]
        + _ACCSPEC + _CNTSPEC
        + [
            pl.BlockSpec((1, D), lambda i: (0, 0)),
            pl.BlockSpec((1, D), lambda i: (0, 0)),
            pl.BlockSpec((D, 2 * D), lambda i: (0, 0)),
            pl.BlockSpec((D, 2 * D), lambda i: (0, 0)),
        ],
        out_specs=[
            pl.BlockSpec((N, D), lambda i: (0, 0)),
            pl.BlockSpec((N, 2 * D), lambda i: (0, 0)),
            pl.BlockSpec((N, 2 * D), lambda i: (0, 0)),
        ],
        out_shape=[
            jax.ShapeDtypeStruct((N, D), F32),
            jax.ShapeDtypeStruct((N, 2 * D), F32),
            jax.ShapeDtypeStruct((N, 2 * D), F32),
        ],
    )(h, acc, acc, cnt, cnt, g.reshape(1, D), be.reshape(1, D), wd, ws)


def _final_body(h_ref, a0_ref, a1_ref, c0_ref, c1_ref, g_ref, be_ref,
                batch_ref, wp_ref, bp_ref, wo_ref, bo_ref, o_ref):
    hn = _update_bn(h_ref, a0_ref, a1_ref, c0_ref, c1_ref, g_ref, be_ref)
    gid = lax.broadcasted_iota(jnp.int32, (G, 1), 0)
    mask = (batch_ref[...] == gid).astype(F32)
    sums = jnp.dot(mask, hn, preferred_element_type=F32)
    cg = jnp.sum(mask, axis=1, keepdims=True)
    p = sums / jnp.maximum(cg, 1.0)
    p = jax.nn.relu(jnp.dot(p, wp_ref[...], preferred_element_type=F32)
                    + bp_ref[...])
    o_ref[...] = (jnp.dot(p, wo_ref[...], preferred_element_type=F32)
                  + bo_ref[...])


def _final(h, acc, cnt, g, be, batch, wp, bp, wo, bo):
    return pl.pallas_call(
        _final_body,
        grid=(1,),
        in_specs=[pl.BlockSpec((N, D), lambda i: (0, 0))]
        + _ACCSPEC + _CNTSPEC
        + [
            pl.BlockSpec((1, D), lambda i: (0, 0)),
            pl.BlockSpec((1, D), lambda i: (0, 0)),
            pl.BlockSpec((1, N), lambda i: (0, 0)),
            pl.BlockSpec((D, D), lambda i: (0, 0)),
            pl.BlockSpec((1, D), lambda i: (0, 0)),
            pl.BlockSpec((D, 1), lambda i: (0, 0)),
            pl.BlockSpec((1, 1), lambda i: (0, 0)),
        ],
        out_specs=pl.BlockSpec((G, 1), lambda i: (0, 0)),
        out_shape=jax.ShapeDtypeStruct((G, 1), F32),
    )(h, acc, acc, cnt, cnt, g.reshape(1, D), be.reshape(1, D),
      batch.reshape(1, N), wp, bp.reshape(1, D), wo, bo.reshape(1, 1))


def _fill(ref, rows, cols, value):
    def row(i, _):
        for q in range(cols // 16):
            ref[i, pl.ds(q * 16, 16)] = jnp.full((16,), value, F32)
        return 0
    lax.fori_loop(0, rows, row, 0)


def _sc_edge_body(dst_hbm, src_hbm, td_hbm, ts_hbm, cfs_hbm, zero_hbm,
                  out_hbm, dsti, srci, gd, gs, cv, mv, acc,
                  sem0, sem1, sem2):
    cid = lax.axis_index("c")
    sid = lax.axis_index("s")
    wid = cid * NSUB + sid

    for r in range(STRIPE // CH):
        pltpu.sync_copy(zero_hbm, acc.at[pl.ds(sid * STRIPE + r * CH, CH)])
    plsc.subcore_barrier()

    def chunk(c, _):
        base = wid * EW + c * CH
        pltpu.sync_copy(dst_hbm.at[pl.ds(base, CH)], dsti)
        pltpu.sync_copy(src_hbm.at[pl.ds(base, CH)], srci)
        cp0 = pltpu.async_copy(td_hbm.at[dsti], gd, sem0)
        cp1 = pltpu.async_copy(ts_hbm.at[srci], gs, sem1)
        cp2 = pltpu.async_copy(cfs_hbm.at[pl.ds(base, CH)], cv, sem2)
        cp0.wait()
        cp1.wait()
        cp2.wait()

        def row(i, _):
            for q in range(D // 16):
                slf = pl.ds(q * 16, 16)
                sls = pl.ds(D + q * 16, 16)
                zf = gd[i, slf] + gs[i, slf] + cv[i, slf]
                zs = gd[i, sls] + gs[i, sls] + cv[i, sls]
                mv[i, slf] = _sigmoid(zf) * _softplus(zs)
            return 0
        lax.fori_loop(0, CH, row, 0)

        pltpu.sync_copy(mv, acc.at[dsti], add=True)
        return 0
    lax.fori_loop(0, NCHUNK, chunk, 0)

    plsc.subcore_barrier()

    pltpu.sync_copy(acc.at[pl.ds(sid * STRIPE, STRIPE)],
                    out_hbm.at[cid, pl.ds(sid * STRIPE, STRIPE)])


_SC_PARAMS = pltpu.CompilerParams(use_tc_tiling_on_sc=False)

_sc_edge = pl.kernel(
    _sc_edge_body,
    out_type=jax.ShapeDtypeStruct((NCORES, NP, D), F32),
    compiler_params=_SC_PARAMS,
    mesh=plsc.VectorSubcoreMesh(core_axis_name="c", subcore_axis_name="s"),
    scratch_types=[
        pltpu.VMEM((CH,), jnp.int32),
        pltpu.VMEM((CH,), jnp.int32),
        pltpu.VMEM((CH, 2 * D), F32),
        pltpu.VMEM((CH, 2 * D), F32),
        pltpu.VMEM((CH, 2 * D), F32),
        pltpu.VMEM((CH, D), F32),
        pltpu.VMEM_SHARED((NP, D), F32),
        pltpu.SemaphoreType.DMA,
        pltpu.SemaphoreType.DMA,
        pltpu.SemaphoreType.DMA,
    ],
)


def _sc_cnt_body(dst_hbm, zero_hbm, out_hbm, dsti, ob, cacc):
    cid = lax.axis_index("c")
    sid = lax.axis_index("s")
    wid = cid * NSUB + sid

    for r in range(STRIPE // CH):
        pltpu.sync_copy(zero_hbm, cacc.at[pl.ds(sid * STRIPE + r * CH, CH)])
    _fill(ob, CH, 16, 1.0)
    plsc.subcore_barrier()

    def chunk(c, _):
        base = wid * EW + c * CH
        pltpu.sync_copy(dst_hbm.at[pl.ds(base, CH)], dsti)
        pltpu.sync_copy(ob, cacc.at[dsti], add=True)
        return 0
    lax.fori_loop(0, NCHUNK, chunk, 0)

    plsc.subcore_barrier()

    pltpu.sync_copy(cacc.at[pl.ds(sid * STRIPE, STRIPE)],
                    out_hbm.at[cid, pl.ds(sid * STRIPE, STRIPE)])


_sc_cnt = pl.kernel(
    _sc_cnt_body,
    out_type=jax.ShapeDtypeStruct((NCORES, NP, 16), F32),
    compiler_params=_SC_PARAMS,
    mesh=plsc.VectorSubcoreMesh(core_axis_name="c", subcore_axis_name="s"),
    scratch_types=[
        pltpu.VMEM((CH,), jnp.int32),
        pltpu.VMEM((CH, 16), F32),
        pltpu.VMEM_SHARED((NP, 16), F32),
    ],
)


def kernel(x, edge_index, edge_attr, batch,
           W_pre, b_pre,
           Wf0, bf0, Ws0, bs0, g0, be0,
           Wf1, bf1, Ws1, bs1, g1, be1,
           Wf2, bf2, Ws2, bs2, g2, be2,
           W_post, b_post, W_out, b_out):
    src = edge_index[0]
    dst = edge_index[1]
    layers = ((Wf0, bf0, Ws0, bs0, g0, be0),
              (Wf1, bf1, Ws1, bs1, g1, be1),
              (Wf2, bf2, Ws2, bs2, g2, be2))

    h = _pre(x, W_pre, b_pre)

    wcat = jnp.stack([
        jnp.concatenate([Wf[2 * D:], Ws[2 * D:]], axis=1)
        for (Wf, _, Ws, _, _, _) in layers])
    bcat = jnp.stack([
        jnp.concatenate([bf, bs]).reshape(1, 2 * D)
        for (_, bf, _, bs, _, _) in layers])
    cfs = _edgeconst(edge_attr, wcat, bcat)

    zero_d = jnp.zeros((CH, D), F32)
    zero_16 = jnp.zeros((CH, 16), F32)
    cnt = _sc_cnt(dst, zero_16)

    acc = None
    for l, (Wf, bf, Ws, bs, g, be) in enumerate(layers):
        wd = jnp.concatenate([Wf[:D], Ws[:D]], axis=1)
        wsrc = jnp.concatenate([Wf[D:2 * D], Ws[D:2 * D]], axis=1)
        if l == 0:
            td, ts = _tables(h, wd, wsrc)
        else:
            h, td, ts = _bn_tables(h, acc, cnt, layers[l - 1][4],
                                   layers[l - 1][5], wd, wsrc)
        acc = _sc_edge(dst, src, td, ts, cfs[l], zero_d)

    return _final(h, acc, cnt, g2, be2, batch,
                  W_post, b_post, W_out, b_out)


# speedup vs baseline: 1.3167x; 1.0847x over previous
import jax
import jax.numpy as jnp
from jax import lax
from jax.experimental import pallas as pl
from jax.experimental.pallas import tpu as pltpu
from jax.experimental.pallas import tpu_sc as plsc

N = 10000
E = 320000
DF = 128
DE = 16
D = 64
G = 64

NCORES = 2
NSUB = 16
NW = NCORES * NSUB
EW = E // NW
CH = 80
NCHUNK = EW // CH
STRIPE = 640
NP = NSUB * STRIPE

F32 = jnp.float32

_L1P = (0.99999997, -0.49999502, 0.33319278, -0.24844407, 0.19111539,
        -0.13674945, 0.07836325, -0.02958924, 0.00525359)


def _sigmoid(x):
    e = jnp.exp(-jnp.abs(x))
    r = 1.0 / (1.0 + e)
    return jnp.where(x >= 0, r, 1.0 - r)


def _softplus(x):
    t = jnp.exp(-jnp.abs(x))
    p = jnp.full(x.shape, _L1P[8], F32)
    for k in range(7, -1, -1):
        p = p * t + _L1P[k]
    return jnp.maximum(x, 0.0) + t * p


def _pre_body(x_ref, w_ref, b_ref, o_ref):
    o_ref[...] = jax.nn.relu(
        jnp.dot(x_ref[...], w_ref[...], preferred_element_type=F32)
        + b_ref[...])


def _pre(x, w, b):
    blk = 1000
    return pl.pallas_call(
        _pre_body,
        grid=(N // blk,),
        in_specs=[
            pl.BlockSpec((blk, DF), lambda i: (i, 0)),
            pl.BlockSpec((DF, D), lambda i: (0, 0)),
            pl.BlockSpec((1, D), lambda i: (0, 0)),
        ],
        out_specs=pl.BlockSpec((blk, D), lambda i: (i, 0)),
        out_shape=jax.ShapeDtypeStruct((N, D), F32),
    )(x, w, b.reshape(1, D))


def _edgeconst_body(ea_ref, w_ref, b_ref, o0_ref, o1_ref, o2_ref):
    ea = ea_ref[...]
    for l, o_ref in enumerate((o0_ref, o1_ref, o2_ref)):
        o_ref[...] = (
            jnp.dot(ea, w_ref[l], preferred_element_type=F32) + b_ref[l])


def _edgeconst(ea, wcat, bcat):
    blk = 4000
    return pl.pallas_call(
        _edgeconst_body,
        grid=(E // blk,),
        in_specs=[
            pl.BlockSpec((blk, DE), lambda i: (i, 0)),
            pl.BlockSpec((3, DE, 2 * D), lambda i: (0, 0, 0)),
            pl.BlockSpec((3, 1, 2 * D), lambda i: (0, 0, 0)),
        ],
        out_specs=[pl.BlockSpec((blk, 2 * D), lambda i: (i, 0))] * 3,
        out_shape=[jax.ShapeDtypeStruct((E, 2 * D), F32)] * 3,
    )(ea, wcat, bcat)


def _tables_body(h_ref, wd_ref, ws_ref, td_ref, ts_ref):
    h = h_ref[...]
    td_ref[...] = jnp.dot(h, wd_ref[...], preferred_element_type=F32)
    ts_ref[...] = jnp.dot(h, ws_ref[...], preferred_element_type=F32)


def _tables(h, wd, ws):
    blk = 1000
    return pl.pallas_call(
        _tables_body,
        grid=(N // blk,),
        in_specs=[
            pl.BlockSpec((blk, D), lambda i: (i, 0)),
            pl.BlockSpec((D, 2 * D), lambda i: (0, 0)),
            pl.BlockSpec((D, 2 * D), lambda i: (0, 0)),
        ],
        out_specs=[pl.BlockSpec((blk, 2 * D), lambda i: (i, 0))] * 2,
        out_shape=[jax.ShapeDtypeStruct((N, 2 * D), F32)] * 2,
    )(h, wd, ws)


def _update_bn(h_ref, a0_ref, a1_ref, c0_ref, c1_ref, g_ref, be_ref):
    cnt = jnp.maximum(c0_ref[0, :, :1] + c1_ref[0, :, :1], 1.0)
    hu = h_ref[...] + (a0_ref[0] + a1_ref[0]) / cnt
    mu = jnp.mean(hu, axis=0, keepdims=True)
    var = jnp.mean((hu - mu) ** 2, axis=0, keepdims=True)
    return (hu - mu) * lax.rsqrt(var + 1e-5) * g_ref[...] + be_ref[...]


def _bn_tables_body(h_ref, a0_ref, a1_ref, c0_ref, c1_ref, g_ref, be_ref,
                    wd_ref, ws_ref, hn_ref, td_ref, ts_ref):
    hn = _update_bn(h_ref, a0_ref, a1_ref, c0_ref, c1_ref, g_ref, be_ref)
    hn_ref[...] = hn
    td_ref[...] = jnp.dot(hn, wd_ref[...], preferred_element_type=F32)
    ts_ref[...] = jnp.dot(hn, ws_ref[...], preferred_element_type=F32)


_ACCSPEC = [
    pl.BlockSpec((1, N, D), lambda i: (0, 0, 0)),
    pl.BlockSpec((1, N, D), lambda i: (1, 0, 0)),
]
_CNTSPEC = [
    pl.BlockSpec((1, N, 16), lambda i: (0, 0, 0)),
    pl.BlockSpec((1, N, 16), lambda i: (1, 0, 0)),
]


def _bn_tables(h, acc, cnt, g, be, wd, ws):
    return pl.pallas_call(
        _bn_tables_body,
        grid=(1,),
        in_specs=[pl.BlockSpec((N, D), lambda i: (0, 0))]
        + _ACCSPEC + _CNTSPEC
        + [
            pl.BlockSpec((1, D), lambda i: (0, 0)),
            pl.BlockSpec((1, D), lambda i: (0, 0)),
            pl.BlockSpec((D, 2 * D), lambda i: (0, 0)),
            pl.BlockSpec((D, 2 * D), lambda i: (0, 0)),
        ],
        out_specs=[
            pl.BlockSpec((N, D), lambda i: (0, 0)),
            pl.BlockSpec((N, 2 * D), lambda i: (0, 0)),
            pl.BlockSpec((N, 2 * D), lambda i: (0, 0)),
        ],
        out_shape=[
            jax.ShapeDtypeStruct((N, D), F32),
            jax.ShapeDtypeStruct((N, 2 * D), F32),
            jax.ShapeDtypeStruct((N, 2 * D), F32),
        ],
    )(h, acc, acc, cnt, cnt, g.reshape(1, D), be.reshape(1, D), wd, ws)


def _final_body(h_ref, a0_ref, a1_ref, c0_ref, c1_ref, g_ref, be_ref,
                batch_ref, wp_ref, bp_ref, wo_ref, bo_ref, o_ref):
    hn = _update_bn(h_ref, a0_ref, a1_ref, c0_ref, c1_ref, g_ref, be_ref)
    gid = lax.broadcasted_iota(jnp.int32, (G, 1), 0)
    mask = (batch_ref[...] == gid).astype(F32)
    sums = jnp.dot(mask, hn, preferred_element_type=F32)
    cg = jnp.sum(mask, axis=1, keepdims=True)
    p = sums / jnp.maximum(cg, 1.0)
    p = jax.nn.relu(jnp.dot(p, wp_ref[...], preferred_element_type=F32)
                    + bp_ref[...])
    o_ref[...] = (jnp.dot(p, wo_ref[...], preferred_element_type=F32)
                  + bo_ref[...])


def _final(h, acc, cnt, g, be, batch, wp, bp, wo, bo):
    return pl.pallas_call(
        _final_body,
        grid=(1,),
        in_specs=[pl.BlockSpec((N, D), lambda i: (0, 0))]
        + _ACCSPEC + _CNTSPEC
        + [
            pl.BlockSpec((1, D), lambda i: (0, 0)),
            pl.BlockSpec((1, D), lambda i: (0, 0)),
            pl.BlockSpec((1, N), lambda i: (0, 0)),
            pl.BlockSpec((D, D), lambda i: (0, 0)),
            pl.BlockSpec((1, D), lambda i: (0, 0)),
            pl.BlockSpec((D, 1), lambda i: (0, 0)),
            pl.BlockSpec((1, 1), lambda i: (0, 0)),
        ],
        out_specs=pl.BlockSpec((G, 1), lambda i: (0, 0)),
        out_shape=jax.ShapeDtypeStruct((G, 1), F32),
    )(h, acc, acc, cnt, cnt, g.reshape(1, D), be.reshape(1, D),
      batch.reshape(1, N), wp, bp.reshape(1, D), wo, bo.reshape(1, 1))


def _fill(ref, rows, cols, value):
    def row(i, _):
        for q in range(cols // 16):
            ref[i, pl.ds(q * 16, 16)] = jnp.full((16,), value, F32)
        return 0
    lax.fori_loop(0, rows, row, 0)


def _sc_edge_body(dst_hbm, src_hbm, td_hbm, ts_hbm, cfs_hbm, zero_hbm,
                  out_hbm, dsti, srci, gd0, gs0, cv0, gd1, gs1, cv1,
                  mv, acc, semd0, sems0, semc0, semd1, sems1, semc1):
    cid = lax.axis_index("c")
    sid = lax.axis_index("s")
    wid = cid * NSUB + sid

    for r in range(STRIPE // CH):
        pltpu.sync_copy(zero_hbm, acc.at[pl.ds(sid * STRIPE + r * CH, CH)])
    pltpu.sync_copy(dst_hbm.at[pl.ds(wid * EW, EW)], dsti)
    pltpu.sync_copy(src_hbm.at[pl.ds(wid * EW, EW)], srci)
    plsc.subcore_barrier()

    bufs = ((gd0, gs0, cv0, semd0, sems0, semc0),
            (gd1, gs1, cv1, semd1, sems1, semc1))

    def issue(c, b):
        gd, gs, cv, sd, ss, sc = bufs[b]
        di = dsti.at[pl.ds(c * CH, CH)]
        si = srci.at[pl.ds(c * CH, CH)]
        return (pltpu.async_copy(td_hbm.at[di], gd, sd),
                pltpu.async_copy(ts_hbm.at[si], gs, ss),
                pltpu.async_copy(cfs_hbm.at[pl.ds(wid * EW + c * CH, CH)],
                                 cv, sc))

    def process(c, b, cps):
        gd, gs, cv = bufs[b][:3]
        for cp in cps:
            cp.wait()

        def row(i, _):
            for q in range(D // 16):
                slf = pl.ds(q * 16, 16)
                sls = pl.ds(D + q * 16, 16)
                zf = gd[i, slf] + gs[i, slf] + cv[i, slf]
                zs = gd[i, sls] + gs[i, sls] + cv[i, sls]
                mv[i, slf] = _sigmoid(zf) * _softplus(zs)
            return 0
        lax.fori_loop(0, CH, row, 0)
        pltpu.sync_copy(mv, acc.at[dsti.at[pl.ds(c * CH, CH)]], add=True)

    def pair(p, _):
        c0 = 2 * p
        cps0 = issue(c0, 0)
        cps1 = issue(c0 + 1, 1)
        process(c0, 0, cps0)
        process(c0 + 1, 1, cps1)
        return 0
    lax.fori_loop(0, NCHUNK // 2, pair, 0)
    if NCHUNK % 2:
        c = NCHUNK - 1
        process(c, 0, issue(c, 0))

    plsc.subcore_barrier()

    pltpu.sync_copy(acc.at[pl.ds(sid * STRIPE, STRIPE)],
                    out_hbm.at[cid, pl.ds(sid * STRIPE, STRIPE)])


_SC_PARAMS = pltpu.CompilerParams(use_tc_tiling_on_sc=False)

_sc_edge = pl.kernel(
    _sc_edge_body,
    out_type=jax.ShapeDtypeStruct((NCORES, NP, D), F32),
    compiler_params=_SC_PARAMS,
    mesh=plsc.VectorSubcoreMesh(core_axis_name="c", subcore_axis_name="s"),
    scratch_types=[
        pltpu.VMEM((EW,), jnp.int32),
        pltpu.VMEM((EW,), jnp.int32),
        pltpu.VMEM((CH, 2 * D), F32),
        pltpu.VMEM((CH, 2 * D), F32),
        pltpu.VMEM((CH, 2 * D), F32),
        pltpu.VMEM((CH, 2 * D), F32),
        pltpu.VMEM((CH, 2 * D), F32),
        pltpu.VMEM((CH, 2 * D), F32),
        pltpu.VMEM((CH, D), F32),
        pltpu.VMEM_SHARED((NP, D), F32),
        pltpu.SemaphoreType.DMA,
        pltpu.SemaphoreType.DMA,
        pltpu.SemaphoreType.DMA,
        pltpu.SemaphoreType.DMA,
        pltpu.SemaphoreType.DMA,
        pltpu.SemaphoreType.DMA,
    ],
)


def _sc_cnt_body(dst_hbm, zero_hbm, out_hbm, dsti, ob, cacc):
    cid = lax.axis_index("c")
    sid = lax.axis_index("s")
    wid = cid * NSUB + sid

    for r in range(STRIPE // CH):
        pltpu.sync_copy(zero_hbm, cacc.at[pl.ds(sid * STRIPE + r * CH, CH)])
    _fill(ob, CH, 16, 1.0)
    plsc.subcore_barrier()

    def chunk(c, _):
        base = wid * EW + c * CH
        pltpu.sync_copy(dst_hbm.at[pl.ds(base, CH)], dsti)
        pltpu.sync_copy(ob, cacc.at[dsti], add=True)
        return 0
    lax.fori_loop(0, NCHUNK, chunk, 0)

    plsc.subcore_barrier()

    pltpu.sync_copy(cacc.at[pl.ds(sid * STRIPE, STRIPE)],
                    out_hbm.at[cid, pl.ds(sid * STRIPE, STRIPE)])


_sc_cnt = pl.kernel(
    _sc_cnt_body,
    out_type=jax.ShapeDtypeStruct((NCORES, NP, 16), F32),
    compiler_params=_SC_PARAMS,
    mesh=plsc.VectorSubcoreMesh(core_axis_name="c", subcore_axis_name="s"),
    scratch_types=[
        pltpu.VMEM((CH,), jnp.int32),
        pltpu.VMEM((CH, 16), F32),
        pltpu.VMEM_SHARED((NP, 16), F32),
    ],
)


def kernel(x, edge_index, edge_attr, batch,
           W_pre, b_pre,
           Wf0, bf0, Ws0, bs0, g0, be0,
           Wf1, bf1, Ws1, bs1, g1, be1,
           Wf2, bf2, Ws2, bs2, g2, be2,
           W_post, b_post, W_out, b_out):
    src = edge_index[0]
    dst = edge_index[1]
    layers = ((Wf0, bf0, Ws0, bs0, g0, be0),
              (Wf1, bf1, Ws1, bs1, g1, be1),
              (Wf2, bf2, Ws2, bs2, g2, be2))

    h = _pre(x, W_pre, b_pre)

    wcat = jnp.stack([
        jnp.concatenate([Wf[2 * D:], Ws[2 * D:]], axis=1)
        for (Wf, _, Ws, _, _, _) in layers])
    bcat = jnp.stack([
        jnp.concatenate([bf, bs]).reshape(1, 2 * D)
        for (_, bf, _, bs, _, _) in layers])
    cfs = _edgeconst(edge_attr, wcat, bcat)

    zero_d = jnp.zeros((CH, D), F32)
    zero_16 = jnp.zeros((CH, 16), F32)
    cnt = _sc_cnt(dst, zero_16)

    acc = None
    for l, (Wf, bf, Ws, bs, g, be) in enumerate(layers):
        wd = jnp.concatenate([Wf[:D], Ws[:D]], axis=1)
        wsrc = jnp.concatenate([Wf[D:2 * D], Ws[D:2 * D]], axis=1)
        if l == 0:
            td, ts = _tables(h, wd, wsrc)
        else:
            h, td, ts = _bn_tables(h, acc, cnt, layers[l - 1][4],
                                   layers[l - 1][5], wd, wsrc)
        acc = _sc_edge(dst, src, td, ts, cfs[l], zero_d)

    return _final(h, acc, cnt, g2, be2, batch,
                  W_post, b_post, W_out, b_out)


# speedup vs baseline: 3.5171x; 2.6712x over previous
import jax
import jax.numpy as jnp
from jax import lax
from jax.experimental import pallas as pl
from jax.experimental.pallas import tpu as pltpu
from jax.experimental.pallas import tpu_sc as plsc

N = 10000
E = 320000
DF = 128
DE = 16
D = 64
G = 64

NCORES = 2
NSUB = 16
NW = NCORES * NSUB
EW = E // NW
CH = 80
NCHUNK = EW // CH
STRIPE = 640
NP = NSUB * STRIPE

F32 = jnp.float32

_L1P = (0.99999997, -0.49999502, 0.33319278, -0.24844407, 0.19111539,
        -0.13674945, 0.07836325, -0.02958924, 0.00525359)


def _sigmoid(x):
    e = jnp.exp(-jnp.abs(x))
    r = 1.0 / (1.0 + e)
    return jnp.where(x >= 0, r, 1.0 - r)


def _softplus(x):
    t = jnp.exp(-jnp.abs(x))
    p = jnp.full(x.shape, _L1P[8], F32)
    for k in range(7, -1, -1):
        p = p * t + _L1P[k]
    return jnp.maximum(x, 0.0) + t * p


def _pre_body(x_ref, w_ref, b_ref, o_ref):
    o_ref[...] = jax.nn.relu(
        jnp.dot(x_ref[...], w_ref[...], preferred_element_type=F32)
        + b_ref[...])


def _pre(x, w, b):
    blk = 1000
    return pl.pallas_call(
        _pre_body,
        grid=(N // blk,),
        in_specs=[
            pl.BlockSpec((blk, DF), lambda i: (i, 0)),
            pl.BlockSpec((DF, D), lambda i: (0, 0)),
            pl.BlockSpec((1, D), lambda i: (0, 0)),
        ],
        out_specs=pl.BlockSpec((blk, D), lambda i: (i, 0)),
        out_shape=jax.ShapeDtypeStruct((N, D), F32),
    )(x, w, b.reshape(1, D))


def _edgeconst_body(ea_ref, w_ref, b_ref, o0_ref, o1_ref, o2_ref):
    ea = ea_ref[...]
    for l, o_ref in enumerate((o0_ref, o1_ref, o2_ref)):
        o_ref[...] = (
            jnp.dot(ea, w_ref[l], preferred_element_type=F32) + b_ref[l])


def _edgeconst(ea, wcat, bcat):
    blk = 4000
    return pl.pallas_call(
        _edgeconst_body,
        grid=(E // blk,),
        in_specs=[
            pl.BlockSpec((blk, DE), lambda i: (i, 0)),
            pl.BlockSpec((3, DE, 2 * D), lambda i: (0, 0, 0)),
            pl.BlockSpec((3, 1, 2 * D), lambda i: (0, 0, 0)),
        ],
        out_specs=[pl.BlockSpec((blk, 2 * D), lambda i: (i, 0))] * 3,
        out_shape=[jax.ShapeDtypeStruct((E, 2 * D), F32)] * 3,
    )(ea, wcat, bcat)


def _tables_body(h_ref, wd_ref, ws_ref, td_ref, ts_ref):
    h = h_ref[...]
    td_ref[...] = jnp.dot(h, wd_ref[...], preferred_element_type=F32)
    ts_ref[...] = jnp.dot(h, ws_ref[...], preferred_element_type=F32)


def _tables(h, wd, ws):
    blk = 1000
    return pl.pallas_call(
        _tables_body,
        grid=(N // blk,),
        in_specs=[
            pl.BlockSpec((blk, D), lambda i: (i, 0)),
            pl.BlockSpec((D, 2 * D), lambda i: (0, 0)),
            pl.BlockSpec((D, 2 * D), lambda i: (0, 0)),
        ],
        out_specs=[pl.BlockSpec((blk, 2 * D), lambda i: (i, 0))] * 2,
        out_shape=[jax.ShapeDtypeStruct((N, 2 * D), F32)] * 2,
    )(h, wd, ws)


def _update_bn(h_ref, a0_ref, a1_ref, c0_ref, c1_ref, g_ref, be_ref):
    cnt = jnp.maximum(c0_ref[0, :, :1] + c1_ref[0, :, :1], 1.0)
    hu = h_ref[...] + (a0_ref[0] + a1_ref[0]) / cnt
    mu = jnp.mean(hu, axis=0, keepdims=True)
    var = jnp.mean((hu - mu) ** 2, axis=0, keepdims=True)
    return (hu - mu) * lax.rsqrt(var + 1e-5) * g_ref[...] + be_ref[...]


def _bn_tables_body(h_ref, a0_ref, a1_ref, c0_ref, c1_ref, g_ref, be_ref,
                    wd_ref, ws_ref, hn_ref, td_ref, ts_ref):
    hn = _update_bn(h_ref, a0_ref, a1_ref, c0_ref, c1_ref, g_ref, be_ref)
    hn_ref[...] = hn
    td_ref[...] = jnp.dot(hn, wd_ref[...], preferred_element_type=F32)
    ts_ref[...] = jnp.dot(hn, ws_ref[...], preferred_element_type=F32)


_ACCSPEC = [
    pl.BlockSpec((1, N, D), lambda i: (0, 0, 0)),
    pl.BlockSpec((1, N, D), lambda i: (1, 0, 0)),
]
_CNTSPEC = [
    pl.BlockSpec((1, N, 16), lambda i: (0, 0, 0)),
    pl.BlockSpec((1, N, 16), lambda i: (1, 0, 0)),
]


def _bn_tables(h, acc, cnt, g, be, wd, ws):
    return pl.pallas_call(
        _bn_tables_body,
        grid=(1,),
        in_specs=[pl.BlockSpec((N, D), lambda i: (0, 0))]
        + _ACCSPEC + _CNTSPEC
        + [
            pl.BlockSpec((1, D), lambda i: (0, 0)),
            pl.BlockSpec((1, D), lambda i: (0, 0)),
            pl.BlockSpec((D, 2 * D), lambda i: (0, 0)),
            pl.BlockSpec((D, 2 * D), lambda i: (0, 0)),
        ],
        out_specs=[
            pl.BlockSpec((N, D), lambda i: (0, 0)),
            pl.BlockSpec((N, 2 * D), lambda i: (0, 0)),
            pl.BlockSpec((N, 2 * D), lambda i: (0, 0)),
        ],
        out_shape=[
            jax.ShapeDtypeStruct((N, D), F32),
            jax.ShapeDtypeStruct((N, 2 * D), F32),
            jax.ShapeDtypeStruct((N, 2 * D), F32),
        ],
    )(h, acc, acc, cnt, cnt, g.reshape(1, D), be.reshape(1, D), wd, ws)


def _final_body(h_ref, a0_ref, a1_ref, c0_ref, c1_ref, g_ref, be_ref,
                batch_ref, wp_ref, bp_ref, wo_ref, bo_ref, o_ref):
    hn = _update_bn(h_ref, a0_ref, a1_ref, c0_ref, c1_ref, g_ref, be_ref)
    gid = lax.broadcasted_iota(jnp.int32, (G, 1), 0)
    mask = (batch_ref[...] == gid).astype(F32)
    sums = jnp.dot(mask, hn, preferred_element_type=F32)
    cg = jnp.sum(mask, axis=1, keepdims=True)
    p = sums / jnp.maximum(cg, 1.0)
    p = jax.nn.relu(jnp.dot(p, wp_ref[...], preferred_element_type=F32)
                    + bp_ref[...])
    o_ref[...] = (jnp.dot(p, wo_ref[...], preferred_element_type=F32)
                  + bo_ref[...])


def _final(h, acc, cnt, g, be, batch, wp, bp, wo, bo):
    return pl.pallas_call(
        _final_body,
        grid=(1,),
        in_specs=[pl.BlockSpec((N, D), lambda i: (0, 0))]
        + _ACCSPEC + _CNTSPEC
        + [
            pl.BlockSpec((1, D), lambda i: (0, 0)),
            pl.BlockSpec((1, D), lambda i: (0, 0)),
            pl.BlockSpec((1, N), lambda i: (0, 0)),
            pl.BlockSpec((D, D), lambda i: (0, 0)),
            pl.BlockSpec((1, D), lambda i: (0, 0)),
            pl.BlockSpec((D, 1), lambda i: (0, 0)),
            pl.BlockSpec((1, 1), lambda i: (0, 0)),
        ],
        out_specs=pl.BlockSpec((G, 1), lambda i: (0, 0)),
        out_shape=jax.ShapeDtypeStruct((G, 1), F32),
    )(h, acc, acc, cnt, cnt, g.reshape(1, D), be.reshape(1, D),
      batch.reshape(1, N), wp, bp.reshape(1, D), wo, bo.reshape(1, 1))


def _fill(ref, rows, cols, value):
    def row(i, _):
        for q in range(cols // 16):
            ref[i, pl.ds(q * 16, 16)] = jnp.full((16,), value, F32)
        return 0
    lax.fori_loop(0, rows, row, 0)


def _sc_edge_body(dst_hbm, src_hbm, td_hbm, ts_hbm, cfs_hbm, zero_hbm,
                  out_hbm, dsti, srci, gd0, gs0, cv0, gd1, gs1, cv1,
                  mv, acc, semd0, sems0, semc0, semd1, sems1, semc1):
    cid = lax.axis_index("c")
    sid = lax.axis_index("s")
    wid = cid * NSUB + sid

    for r in range(STRIPE // CH):
        pltpu.sync_copy(zero_hbm, acc.at[pl.ds(sid * STRIPE + r * CH, CH)])
    pltpu.sync_copy(dst_hbm.at[pl.ds(wid * EW, EW)], dsti)
    pltpu.sync_copy(src_hbm.at[pl.ds(wid * EW, EW)], srci)
    plsc.subcore_barrier()

    bufs = ((gd0, gs0, cv0, semd0, sems0, semc0),
            (gd1, gs1, cv1, semd1, sems1, semc1))

    def issue(c, b):
        gd, gs, cv, sd, ss, sc = bufs[b]
        di = dsti.at[pl.ds(c * CH, CH)]
        si = srci.at[pl.ds(c * CH, CH)]
        return (pltpu.async_copy(td_hbm.at[di], gd, sd),
                pltpu.async_copy(ts_hbm.at[si], gs, ss),
                pltpu.async_copy(cfs_hbm.at[pl.ds(wid * EW + c * CH, CH)],
                                 cv, sc))

    def process(c, b, cps):
        gd, gs, cv = bufs[b][:3]
        for cp in cps:
            cp.wait()

        @plsc.parallel_loop(0, CH, unroll=2)
        def row(i):
            for q in range(D // 16):
                slf = pl.ds(q * 16, 16)
                sls = pl.ds(D + q * 16, 16)
                zf = gd[i, slf] + gs[i, slf] + cv[i, slf]
                zs = gd[i, sls] + gs[i, sls] + cv[i, sls]
                mv[i, slf] = _sigmoid(zf) * _softplus(zs)
        pltpu.sync_copy(mv, acc.at[dsti.at[pl.ds(c * CH, CH)]], add=True)

    def pair(p, _):
        c0 = 2 * p
        cps0 = issue(c0, 0)
        cps1 = issue(c0 + 1, 1)
        process(c0, 0, cps0)
        process(c0 + 1, 1, cps1)
        return 0
    lax.fori_loop(0, NCHUNK // 2, pair, 0)
    if NCHUNK % 2:
        c = NCHUNK - 1
        process(c, 0, issue(c, 0))

    plsc.subcore_barrier()

    pltpu.sync_copy(acc.at[pl.ds(sid * STRIPE, STRIPE)],
                    out_hbm.at[cid, pl.ds(sid * STRIPE, STRIPE)])


_SC_PARAMS = pltpu.CompilerParams(use_tc_tiling_on_sc=False)

_sc_edge = pl.kernel(
    _sc_edge_body,
    out_type=jax.ShapeDtypeStruct((NCORES, NP, D), F32),
    compiler_params=_SC_PARAMS,
    mesh=plsc.VectorSubcoreMesh(core_axis_name="c", subcore_axis_name="s"),
    scratch_types=[
        pltpu.VMEM((EW,), jnp.int32),
        pltpu.VMEM((EW,), jnp.int32),
        pltpu.VMEM((CH, 2 * D), F32),
        pltpu.VMEM((CH, 2 * D), F32),
        pltpu.VMEM((CH, 2 * D), F32),
        pltpu.VMEM((CH, 2 * D), F32),
        pltpu.VMEM((CH, 2 * D), F32),
        pltpu.VMEM((CH, 2 * D), F32),
        pltpu.VMEM((CH, D), F32),
        pltpu.VMEM_SHARED((NP, D), F32),
        pltpu.SemaphoreType.DMA,
        pltpu.SemaphoreType.DMA,
        pltpu.SemaphoreType.DMA,
        pltpu.SemaphoreType.DMA,
        pltpu.SemaphoreType.DMA,
        pltpu.SemaphoreType.DMA,
    ],
)


def _sc_cnt_body(dst_hbm, zero_hbm, out_hbm, dsti, ob, cacc):
    cid = lax.axis_index("c")
    sid = lax.axis_index("s")
    wid = cid * NSUB + sid

    for r in range(STRIPE // CH):
        pltpu.sync_copy(zero_hbm, cacc.at[pl.ds(sid * STRIPE + r * CH, CH)])
    _fill(ob, CH, 16, 1.0)
    plsc.subcore_barrier()

    def chunk(c, _):
        base = wid * EW + c * CH
        pltpu.sync_copy(dst_hbm.at[pl.ds(base, CH)], dsti)
        pltpu.sync_copy(ob, cacc.at[dsti], add=True)
        return 0
    lax.fori_loop(0, NCHUNK, chunk, 0)

    plsc.subcore_barrier()

    pltpu.sync_copy(cacc.at[pl.ds(sid * STRIPE, STRIPE)],
                    out_hbm.at[cid, pl.ds(sid * STRIPE, STRIPE)])


_sc_cnt = pl.kernel(
    _sc_cnt_body,
    out_type=jax.ShapeDtypeStruct((NCORES, NP, 16), F32),
    compiler_params=_SC_PARAMS,
    mesh=plsc.VectorSubcoreMesh(core_axis_name="c", subcore_axis_name="s"),
    scratch_types=[
        pltpu.VMEM((CH,), jnp.int32),
        pltpu.VMEM((CH, 16), F32),
        pltpu.VMEM_SHARED((NP, 16), F32),
    ],
)


def kernel(x, edge_index, edge_attr, batch,
           W_pre, b_pre,
           Wf0, bf0, Ws0, bs0, g0, be0,
           Wf1, bf1, Ws1, bs1, g1, be1,
           Wf2, bf2, Ws2, bs2, g2, be2,
           W_post, b_post, W_out, b_out):
    src = edge_index[0]
    dst = edge_index[1]
    layers = ((Wf0, bf0, Ws0, bs0, g0, be0),
              (Wf1, bf1, Ws1, bs1, g1, be1),
              (Wf2, bf2, Ws2, bs2, g2, be2))

    h = _pre(x, W_pre, b_pre)

    wcat = jnp.stack([
        jnp.concatenate([Wf[2 * D:], Ws[2 * D:]], axis=1)
        for (Wf, _, Ws, _, _, _) in layers])
    bcat = jnp.stack([
        jnp.concatenate([bf, bs]).reshape(1, 2 * D)
        for (_, bf, _, bs, _, _) in layers])
    cfs = _edgeconst(edge_attr, wcat, bcat)

    zero_d = jnp.zeros((CH, D), F32)
    zero_16 = jnp.zeros((CH, 16), F32)
    cnt = _sc_cnt(dst, zero_16)

    acc = None
    for l, (Wf, bf, Ws, bs, g, be) in enumerate(layers):
        wd = jnp.concatenate([Wf[:D], Ws[:D]], axis=1)
        wsrc = jnp.concatenate([Wf[D:2 * D], Ws[D:2 * D]], axis=1)
        if l == 0:
            td, ts = _tables(h, wd, wsrc)
        else:
            h, td, ts = _bn_tables(h, acc, cnt, layers[l - 1][4],
                                   layers[l - 1][5], wd, wsrc)
        acc = _sc_edge(dst, src, td, ts, cfs[l], zero_d)

    return _final(h, acc, cnt, g2, be2, batch,
                  W_post, b_post, W_out, b_out)


# speedup vs baseline: 3.5292x; 1.0034x over previous
import jax
import jax.numpy as jnp
from jax import lax
from jax.experimental import pallas as pl
from jax.experimental.pallas import tpu as pltpu
from jax.experimental.pallas import tpu_sc as plsc

N = 10000
E = 320000
DF = 128
DE = 16
D = 64
G = 64

NCORES = 2
NSUB = 16
NW = NCORES * NSUB
EW = E // NW
CH = 80
NCHUNK = EW // CH
STRIPE = 640
NP = NSUB * STRIPE

F32 = jnp.float32

_L1P = (0.99999997, -0.49999502, 0.33319278, -0.24844407, 0.19111539,
        -0.13674945, 0.07836325, -0.02958924, 0.00525359)


def _sigmoid(x):
    e = jnp.exp(-jnp.abs(x))
    r = 1.0 / (1.0 + e)
    return jnp.where(x >= 0, r, 1.0 - r)


def _softplus(x):
    t = jnp.exp(-jnp.abs(x))
    p = jnp.full(x.shape, _L1P[8], F32)
    for k in range(7, -1, -1):
        p = p * t + _L1P[k]
    return jnp.maximum(x, 0.0) + t * p


def _pre_body(x_ref, w_ref, b_ref, o_ref):
    o_ref[...] = jax.nn.relu(
        jnp.dot(x_ref[...], w_ref[...], preferred_element_type=F32)
        + b_ref[...])


def _pre(x, w, b):
    blk = 1000
    return pl.pallas_call(
        _pre_body,
        grid=(N // blk,),
        in_specs=[
            pl.BlockSpec((blk, DF), lambda i: (i, 0)),
            pl.BlockSpec((DF, D), lambda i: (0, 0)),
            pl.BlockSpec((1, D), lambda i: (0, 0)),
        ],
        out_specs=pl.BlockSpec((blk, D), lambda i: (i, 0)),
        out_shape=jax.ShapeDtypeStruct((N, D), F32),
    )(x, w, b.reshape(1, D))


def _edgeconst_body(ea_ref, w_ref, b_ref, o0_ref, o1_ref, o2_ref):
    ea = ea_ref[...]
    for l, o_ref in enumerate((o0_ref, o1_ref, o2_ref)):
        o_ref[...] = (
            jnp.dot(ea, w_ref[l], preferred_element_type=F32) + b_ref[l])


def _edgeconst(ea, wcat, bcat):
    blk = 4000
    return pl.pallas_call(
        _edgeconst_body,
        grid=(E // blk,),
        in_specs=[
            pl.BlockSpec((blk, DE), lambda i: (i, 0)),
            pl.BlockSpec((3, DE, 2 * D), lambda i: (0, 0, 0)),
            pl.BlockSpec((3, 1, 2 * D), lambda i: (0, 0, 0)),
        ],
        out_specs=[pl.BlockSpec((blk, 2 * D), lambda i: (i, 0))] * 3,
        out_shape=[jax.ShapeDtypeStruct((E, 2 * D), F32)] * 3,
    )(ea, wcat, bcat)


def _tables_body(h_ref, wd_ref, ws_ref, td_ref, ts_ref):
    h = h_ref[...]
    td_ref[...] = jnp.dot(h, wd_ref[...], preferred_element_type=F32)
    ts_ref[...] = jnp.dot(h, ws_ref[...], preferred_element_type=F32)


def _tables(h, wd, ws):
    blk = 1000
    return pl.pallas_call(
        _tables_body,
        grid=(N // blk,),
        in_specs=[
            pl.BlockSpec((blk, D), lambda i: (i, 0)),
            pl.BlockSpec((D, 2 * D), lambda i: (0, 0)),
            pl.BlockSpec((D, 2 * D), lambda i: (0, 0)),
        ],
        out_specs=[pl.BlockSpec((blk, 2 * D), lambda i: (i, 0))] * 2,
        out_shape=[jax.ShapeDtypeStruct((N, 2 * D), F32)] * 2,
    )(h, wd, ws)


def _update_bn(h_ref, a0_ref, a1_ref, c0_ref, c1_ref, g_ref, be_ref):
    cnt = jnp.maximum(c0_ref[0, :, :1] + c1_ref[0, :, :1], 1.0)
    hu = h_ref[...] + (a0_ref[0] + a1_ref[0]) / cnt
    mu = jnp.mean(hu, axis=0, keepdims=True)
    var = jnp.mean((hu - mu) ** 2, axis=0, keepdims=True)
    return (hu - mu) * lax.rsqrt(var + 1e-5) * g_ref[...] + be_ref[...]


def _bn_tables_body(h_ref, a0_ref, a1_ref, c0_ref, c1_ref, g_ref, be_ref,
                    wd_ref, ws_ref, hn_ref, td_ref, ts_ref):
    hn = _update_bn(h_ref, a0_ref, a1_ref, c0_ref, c1_ref, g_ref, be_ref)
    hn_ref[...] = hn
    td_ref[...] = jnp.dot(hn, wd_ref[...], preferred_element_type=F32)
    ts_ref[...] = jnp.dot(hn, ws_ref[...], preferred_element_type=F32)


_ACCSPEC = [
    pl.BlockSpec((1, N, D), lambda i: (0, 0, 0)),
    pl.BlockSpec((1, N, D), lambda i: (1, 0, 0)),
]
_CNTSPEC = [
    pl.BlockSpec((1, N, 16), lambda i: (0, 0, 0)),
    pl.BlockSpec((1, N, 16), lambda i: (1, 0, 0)),
]


def _bn_tables(h, acc, cnt, g, be, wd, ws):
    return pl.pallas_call(
        _bn_tables_body,
        grid=(1,),
        in_specs=[pl.BlockSpec((N, D), lambda i: (0, 0))]
        + _ACCSPEC + _CNTSPEC
        + [
            pl.BlockSpec((1, D), lambda i: (0, 0)),
            pl.BlockSpec((1, D), lambda i: (0, 0)),
            pl.BlockSpec((D, 2 * D), lambda i: (0, 0)),
            pl.BlockSpec((D, 2 * D), lambda i: (0, 0)),
        ],
        out_specs=[
            pl.BlockSpec((N, D), lambda i: (0, 0)),
            pl.BlockSpec((N, 2 * D), lambda i: (0, 0)),
            pl.BlockSpec((N, 2 * D), lambda i: (0, 0)),
        ],
        out_shape=[
            jax.ShapeDtypeStruct((N, D), F32),
            jax.ShapeDtypeStruct((N, 2 * D), F32),
            jax.ShapeDtypeStruct((N, 2 * D), F32),
        ],
    )(h, acc, acc, cnt, cnt, g.reshape(1, D), be.reshape(1, D), wd, ws)


def _final_body(h_ref, a0_ref, a1_ref, c0_ref, c1_ref, g_ref, be_ref,
                batch_ref, wp_ref, bp_ref, wo_ref, bo_ref, o_ref):
    hn = _update_bn(h_ref, a0_ref, a1_ref, c0_ref, c1_ref, g_ref, be_ref)
    gid = lax.broadcasted_iota(jnp.int32, (G, 1), 0)
    mask = (batch_ref[...] == gid).astype(F32)
    sums = jnp.dot(mask, hn, preferred_element_type=F32)
    cg = jnp.sum(mask, axis=1, keepdims=True)
    p = sums / jnp.maximum(cg, 1.0)
    p = jax.nn.relu(jnp.dot(p, wp_ref[...], preferred_element_type=F32)
                    + bp_ref[...])
    o_ref[...] = (jnp.dot(p, wo_ref[...], preferred_element_type=F32)
                  + bo_ref[...])


def _final(h, acc, cnt, g, be, batch, wp, bp, wo, bo):
    return pl.pallas_call(
        _final_body,
        grid=(1,),
        in_specs=[pl.BlockSpec((N, D), lambda i: (0, 0))]
        + _ACCSPEC + _CNTSPEC
        + [
            pl.BlockSpec((1, D), lambda i: (0, 0)),
            pl.BlockSpec((1, D), lambda i: (0, 0)),
            pl.BlockSpec((1, N), lambda i: (0, 0)),
            pl.BlockSpec((D, D), lambda i: (0, 0)),
            pl.BlockSpec((1, D), lambda i: (0, 0)),
            pl.BlockSpec((D, 1), lambda i: (0, 0)),
            pl.BlockSpec((1, 1), lambda i: (0, 0)),
        ],
        out_specs=pl.BlockSpec((G, 1), lambda i: (0, 0)),
        out_shape=jax.ShapeDtypeStruct((G, 1), F32),
    )(h, acc, acc, cnt, cnt, g.reshape(1, D), be.reshape(1, D),
      batch.reshape(1, N), wp, bp.reshape(1, D), wo, bo.reshape(1, 1))


def _fill(ref, rows, cols, value):
    def row(i, _):
        for q in range(cols // 16):
            ref[i, pl.ds(q * 16, 16)] = jnp.full((16,), value, F32)
        return 0
    lax.fori_loop(0, rows, row, 0)


def _sc_edge_body(dst_hbm, src_hbm, td_hbm, ts_hbm, cfs_hbm, zero_hbm,
                  out_hbm, dsti, srci, gd0, gs0, cv0, gd1, gs1, cv1,
                  mv, acc, semd0, sems0, semc0, semd1, sems1, semc1):
    cid = lax.axis_index("c")
    sid = lax.axis_index("s")
    wid = cid * NSUB + sid

    for r in range(STRIPE // CH):
        pltpu.sync_copy(zero_hbm, acc.at[pl.ds(sid * STRIPE + r * CH, CH)])
    pltpu.sync_copy(dst_hbm.at[pl.ds(wid * EW, EW)], dsti)
    pltpu.sync_copy(src_hbm.at[pl.ds(wid * EW, EW)], srci)
    plsc.subcore_barrier()

    bufs = ((gd0, gs0, cv0, semd0, sems0, semc0),
            (gd1, gs1, cv1, semd1, sems1, semc1))

    def issue(c, b):
        gd, gs, cv, sd, ss, sc = bufs[b]
        di = dsti.at[pl.ds(c * CH, CH)]
        si = srci.at[pl.ds(c * CH, CH)]
        return (pltpu.async_copy(td_hbm.at[di], gd, sd),
                pltpu.async_copy(ts_hbm.at[si], gs, ss),
                pltpu.async_copy(cfs_hbm.at[pl.ds(wid * EW + c * CH, CH)],
                                 cv, sc))

    def process(c, b, cps):
        gd, gs, cv = bufs[b][:3]
        for cp in cps:
            cp.wait()

        @plsc.parallel_loop(0, CH, unroll=4)
        def row(i):
            for q in range(D // 16):
                slf = pl.ds(q * 16, 16)
                sls = pl.ds(D + q * 16, 16)
                zf = gd[i, slf] + gs[i, slf] + cv[i, slf]
                zs = gd[i, sls] + gs[i, sls] + cv[i, sls]
                mv[i, slf] = _sigmoid(zf) * _softplus(zs)
        pltpu.sync_copy(mv, acc.at[dsti.at[pl.ds(c * CH, CH)]], add=True)

    def pair(p, _):
        c0 = 2 * p
        cps0 = issue(c0, 0)
        cps1 = issue(c0 + 1, 1)
        process(c0, 0, cps0)
        process(c0 + 1, 1, cps1)
        return 0
    lax.fori_loop(0, NCHUNK // 2, pair, 0)
    if NCHUNK % 2:
        c = NCHUNK - 1
        process(c, 0, issue(c, 0))

    plsc.subcore_barrier()

    pltpu.sync_copy(acc.at[pl.ds(sid * STRIPE, STRIPE)],
                    out_hbm.at[cid, pl.ds(sid * STRIPE, STRIPE)])


_SC_PARAMS = pltpu.CompilerParams(use_tc_tiling_on_sc=False)

_sc_edge = pl.kernel(
    _sc_edge_body,
    out_type=jax.ShapeDtypeStruct((NCORES, NP, D), F32),
    compiler_params=_SC_PARAMS,
    mesh=plsc.VectorSubcoreMesh(core_axis_name="c", subcore_axis_name="s"),
    scratch_types=[
        pltpu.VMEM((EW,), jnp.int32),
        pltpu.VMEM((EW,), jnp.int32),
        pltpu.VMEM((CH, 2 * D), F32),
        pltpu.VMEM((CH, 2 * D), F32),
        pltpu.VMEM((CH, 2 * D), F32),
        pltpu.VMEM((CH, 2 * D), F32),
        pltpu.VMEM((CH, 2 * D), F32),
        pltpu.VMEM((CH, 2 * D), F32),
        pltpu.VMEM((CH, D), F32),
        pltpu.VMEM_SHARED((NP, D), F32),
        pltpu.SemaphoreType.DMA,
        pltpu.SemaphoreType.DMA,
        pltpu.SemaphoreType.DMA,
        pltpu.SemaphoreType.DMA,
        pltpu.SemaphoreType.DMA,
        pltpu.SemaphoreType.DMA,
    ],
)


def _sc_cnt_body(dst_hbm, zero_hbm, out_hbm, dsti, ob, cacc):
    cid = lax.axis_index("c")
    sid = lax.axis_index("s")
    wid = cid * NSUB + sid

    for r in range(STRIPE // CH):
        pltpu.sync_copy(zero_hbm, cacc.at[pl.ds(sid * STRIPE + r * CH, CH)])
    _fill(ob, CH, 16, 1.0)
    plsc.subcore_barrier()

    def chunk(c, _):
        base = wid * EW + c * CH
        pltpu.sync_copy(dst_hbm.at[pl.ds(base, CH)], dsti)
        pltpu.sync_copy(ob, cacc.at[dsti], add=True)
        return 0
    lax.fori_loop(0, NCHUNK, chunk, 0)

    plsc.subcore_barrier()

    pltpu.sync_copy(cacc.at[pl.ds(sid * STRIPE, STRIPE)],
                    out_hbm.at[cid, pl.ds(sid * STRIPE, STRIPE)])


_sc_cnt = pl.kernel(
    _sc_cnt_body,
    out_type=jax.ShapeDtypeStruct((NCORES, NP, 16), F32),
    compiler_params=_SC_PARAMS,
    mesh=plsc.VectorSubcoreMesh(core_axis_name="c", subcore_axis_name="s"),
    scratch_types=[
        pltpu.VMEM((CH,), jnp.int32),
        pltpu.VMEM((CH, 16), F32),
        pltpu.VMEM_SHARED((NP, 16), F32),
    ],
)


def kernel(x, edge_index, edge_attr, batch,
           W_pre, b_pre,
           Wf0, bf0, Ws0, bs0, g0, be0,
           Wf1, bf1, Ws1, bs1, g1, be1,
           Wf2, bf2, Ws2, bs2, g2, be2,
           W_post, b_post, W_out, b_out):
    src = edge_index[0]
    dst = edge_index[1]
    layers = ((Wf0, bf0, Ws0, bs0, g0, be0),
              (Wf1, bf1, Ws1, bs1, g1, be1),
              (Wf2, bf2, Ws2, bs2, g2, be2))

    h = _pre(x, W_pre, b_pre)

    wcat = jnp.stack([
        jnp.concatenate([Wf[2 * D:], Ws[2 * D:]], axis=1)
        for (Wf, _, Ws, _, _, _) in layers])
    bcat = jnp.stack([
        jnp.concatenate([bf, bs]).reshape(1, 2 * D)
        for (_, bf, _, bs, _, _) in layers])
    cfs = _edgeconst(edge_attr, wcat, bcat)

    zero_d = jnp.zeros((CH, D), F32)
    zero_16 = jnp.zeros((CH, 16), F32)
    cnt = _sc_cnt(dst, zero_16)

    acc = None
    for l, (Wf, bf, Ws, bs, g, be) in enumerate(layers):
        wd = jnp.concatenate([Wf[:D], Ws[:D]], axis=1)
        wsrc = jnp.concatenate([Wf[D:2 * D], Ws[D:2 * D]], axis=1)
        if l == 0:
            td, ts = _tables(h, wd, wsrc)
        else:
            h, td, ts = _bn_tables(h, acc, cnt, layers[l - 1][4],
                                   layers[l - 1][5], wd, wsrc)
        acc = _sc_edge(dst, src, td, ts, cfs[l], zero_d)

    return _final(h, acc, cnt, g2, be2, batch,
                  W_post, b_post, W_out, b_out)


# speedup vs baseline: 4.6724x; 1.3239x over previous
import jax
import jax.numpy as jnp
from jax import lax
from jax.experimental import pallas as pl
from jax.experimental.pallas import tpu as pltpu
from jax.experimental.pallas import tpu_sc as plsc

N = 10000
E = 320000
DF = 128
DE = 16
D = 64
G = 64

NCORES = 2
NSUB = 16
NW = NCORES * NSUB
EW = E // NW
CH = 80
NCHUNK = EW // CH
STRIPE = 640
NP = NSUB * STRIPE

F32 = jnp.float32

_L1P = (0.99999997, -0.49999502, 0.33319278, -0.24844407, 0.19111539,
        -0.13674945, 0.07836325, -0.02958924, 0.00525359)


def _sigmoid(x):
    e = jnp.exp(-jnp.abs(x))
    r = 1.0 / (1.0 + e)
    return jnp.where(x >= 0, r, 1.0 - r)


def _softplus(x):
    t = jnp.exp(-jnp.abs(x))
    p = jnp.full(x.shape, _L1P[8], F32)
    for k in range(7, -1, -1):
        p = p * t + _L1P[k]
    return jnp.maximum(x, 0.0) + t * p


def _pre_body(x_ref, w_ref, b_ref, o_ref):
    o_ref[...] = jax.nn.relu(
        jnp.dot(x_ref[...], w_ref[...], preferred_element_type=F32)
        + b_ref[...])


def _pre(x, w, b):
    blk = 1000
    return pl.pallas_call(
        _pre_body,
        grid=(N // blk,),
        in_specs=[
            pl.BlockSpec((blk, DF), lambda i: (i, 0)),
            pl.BlockSpec((DF, D), lambda i: (0, 0)),
            pl.BlockSpec((1, D), lambda i: (0, 0)),
        ],
        out_specs=pl.BlockSpec((blk, D), lambda i: (i, 0)),
        out_shape=jax.ShapeDtypeStruct((N, D), F32),
    )(x, w, b.reshape(1, D))


def _edgeconst_body(ea_ref, w_ref, b_ref, o0_ref, o1_ref, o2_ref):
    ea = ea_ref[...]
    for l, o_ref in enumerate((o0_ref, o1_ref, o2_ref)):
        o_ref[...] = (
            jnp.dot(ea, w_ref[l], preferred_element_type=F32) + b_ref[l])


def _edgeconst(ea, wcat, bcat):
    blk = 4000
    return pl.pallas_call(
        _edgeconst_body,
        grid=(E // blk,),
        in_specs=[
            pl.BlockSpec((blk, DE), lambda i: (i, 0)),
            pl.BlockSpec((3, DE, 2 * D), lambda i: (0, 0, 0)),
            pl.BlockSpec((3, 1, 2 * D), lambda i: (0, 0, 0)),
        ],
        out_specs=[pl.BlockSpec((blk, 2 * D), lambda i: (i, 0))] * 3,
        out_shape=[jax.ShapeDtypeStruct((E, 2 * D), F32)] * 3,
    )(ea, wcat, bcat)


def _tables_body(h_ref, wd_ref, ws_ref, td_ref, ts_ref):
    h = h_ref[...]
    td_ref[...] = jnp.dot(h, wd_ref[...], preferred_element_type=F32)
    ts_ref[...] = jnp.dot(h, ws_ref[...], preferred_element_type=F32)


def _tables(h, wd, ws):
    blk = 1000
    return pl.pallas_call(
        _tables_body,
        grid=(N // blk,),
        in_specs=[
            pl.BlockSpec((blk, D), lambda i: (i, 0)),
            pl.BlockSpec((D, 2 * D), lambda i: (0, 0)),
            pl.BlockSpec((D, 2 * D), lambda i: (0, 0)),
        ],
        out_specs=[pl.BlockSpec((blk, 2 * D), lambda i: (i, 0))] * 2,
        out_shape=[jax.ShapeDtypeStruct((N, 2 * D), F32)] * 2,
    )(h, wd, ws)


def _update_bn(h_ref, a0_ref, a1_ref, c0_ref, c1_ref, g_ref, be_ref):
    cnt = jnp.maximum(c0_ref[0, :, :1] + c1_ref[0, :, :1], 1.0)
    hu = h_ref[...] + (a0_ref[0] + a1_ref[0]) / cnt
    mu = jnp.mean(hu, axis=0, keepdims=True)
    var = jnp.mean((hu - mu) ** 2, axis=0, keepdims=True)
    return (hu - mu) * lax.rsqrt(var + 1e-5) * g_ref[...] + be_ref[...]


def _bn_tables_body(h_ref, a0_ref, a1_ref, c0_ref, c1_ref, g_ref, be_ref,
                    wd_ref, ws_ref, hn_ref, td_ref, ts_ref):
    hn = _update_bn(h_ref, a0_ref, a1_ref, c0_ref, c1_ref, g_ref, be_ref)
    hn_ref[...] = hn
    td_ref[...] = jnp.dot(hn, wd_ref[...], preferred_element_type=F32)
    ts_ref[...] = jnp.dot(hn, ws_ref[...], preferred_element_type=F32)


_ACCSPEC = [
    pl.BlockSpec((1, N, D), lambda i: (0, 0, 0)),
    pl.BlockSpec((1, N, D), lambda i: (1, 0, 0)),
]
_CNTSPEC = [
    pl.BlockSpec((1, N, 16), lambda i: (0, 0, 0)),
    pl.BlockSpec((1, N, 16), lambda i: (1, 0, 0)),
]


def _bn_tables(h, acc, cnt, g, be, wd, ws):
    return pl.pallas_call(
        _bn_tables_body,
        grid=(1,),
        in_specs=[pl.BlockSpec((N, D), lambda i: (0, 0))]
        + _ACCSPEC + _CNTSPEC
        + [
            pl.BlockSpec((1, D), lambda i: (0, 0)),
            pl.BlockSpec((1, D), lambda i: (0, 0)),
            pl.BlockSpec((D, 2 * D), lambda i: (0, 0)),
            pl.BlockSpec((D, 2 * D), lambda i: (0, 0)),
        ],
        out_specs=[
            pl.BlockSpec((N, D), lambda i: (0, 0)),
            pl.BlockSpec((N, 2 * D), lambda i: (0, 0)),
            pl.BlockSpec((N, 2 * D), lambda i: (0, 0)),
        ],
        out_shape=[
            jax.ShapeDtypeStruct((N, D), F32),
            jax.ShapeDtypeStruct((N, 2 * D), F32),
            jax.ShapeDtypeStruct((N, 2 * D), F32),
        ],
    )(h, acc, acc, cnt, cnt, g.reshape(1, D), be.reshape(1, D), wd, ws)


def _final_body(h_ref, a0_ref, a1_ref, c0_ref, c1_ref, g_ref, be_ref,
                batch_ref, wp_ref, bp_ref, wo_ref, bo_ref, o_ref):
    hn = _update_bn(h_ref, a0_ref, a1_ref, c0_ref, c1_ref, g_ref, be_ref)
    gid = lax.broadcasted_iota(jnp.int32, (G, 1), 0)
    mask = (batch_ref[...] == gid).astype(F32)
    sums = jnp.dot(mask, hn, preferred_element_type=F32)
    cg = jnp.sum(mask, axis=1, keepdims=True)
    p = sums / jnp.maximum(cg, 1.0)
    p = jax.nn.relu(jnp.dot(p, wp_ref[...], preferred_element_type=F32)
                    + bp_ref[...])
    o_ref[...] = (jnp.dot(p, wo_ref[...], preferred_element_type=F32)
                  + bo_ref[...])


def _final(h, acc, cnt, g, be, batch, wp, bp, wo, bo):
    return pl.pallas_call(
        _final_body,
        grid=(1,),
        in_specs=[pl.BlockSpec((N, D), lambda i: (0, 0))]
        + _ACCSPEC + _CNTSPEC
        + [
            pl.BlockSpec((1, D), lambda i: (0, 0)),
            pl.BlockSpec((1, D), lambda i: (0, 0)),
            pl.BlockSpec((1, N), lambda i: (0, 0)),
            pl.BlockSpec((D, D), lambda i: (0, 0)),
            pl.BlockSpec((1, D), lambda i: (0, 0)),
            pl.BlockSpec((D, 1), lambda i: (0, 0)),
            pl.BlockSpec((1, 1), lambda i: (0, 0)),
        ],
        out_specs=pl.BlockSpec((G, 1), lambda i: (0, 0)),
        out_shape=jax.ShapeDtypeStruct((G, 1), F32),
    )(h, acc, acc, cnt, cnt, g.reshape(1, D), be.reshape(1, D),
      batch.reshape(1, N), wp, bp.reshape(1, D), wo, bo.reshape(1, 1))


def _fill(ref, rows, cols, value):
    def row(i, _):
        for q in range(cols // 16):
            ref[i, pl.ds(q * 16, 16)] = jnp.full((16,), value, F32)
        return 0
    lax.fori_loop(0, rows, row, 0)


def _sc_edge_body(dst_hbm, src_hbm, td_hbm, ts_hbm, cfs_hbm, zero_hbm,
                  out_hbm, dsti, srci, gd0, gs0, cv0, gd1, gs1, cv1,
                  mv, acc, semd0, sems0, semc0, semd1, sems1, semc1):
    cid = lax.axis_index("c")
    sid = lax.axis_index("s")
    wid = cid * NSUB + sid

    for r in range(STRIPE // CH):
        pltpu.sync_copy(zero_hbm, acc.at[pl.ds(sid * STRIPE + r * CH, CH)])
    pltpu.sync_copy(dst_hbm.at[pl.ds(wid * EW, EW)], dsti)
    pltpu.sync_copy(src_hbm.at[pl.ds(wid * EW, EW)], srci)
    plsc.subcore_barrier()

    bufs = ((gd0, gs0, cv0, semd0, sems0, semc0),
            (gd1, gs1, cv1, semd1, sems1, semc1))

    def _copies(c, b):
        gd, gs, cv, sd, ss, sc = bufs[b]
        di = dsti.at[pl.ds(c * CH, CH)]
        si = srci.at[pl.ds(c * CH, CH)]
        return ((td_hbm.at[di], gd, sd),
                (ts_hbm.at[si], gs, ss),
                (cfs_hbm.at[pl.ds(wid * EW + c * CH, CH)], cv, sc))

    def issue(c, b):
        for src, dstb, sem in _copies(c, b):
            pltpu.async_copy(src, dstb, sem)

    def wait(c, b):
        for src, dstb, sem in _copies(c, b):
            pltpu.make_async_copy(src, dstb, sem).wait()

    def compute_scatter(c, b):
        gd, gs, cv = bufs[b][:3]

        @plsc.parallel_loop(0, CH, unroll=4)
        def row(i):
            for q in range(D // 16):
                slf = pl.ds(q * 16, 16)
                sls = pl.ds(D + q * 16, 16)
                zf = gd[i, slf] + gs[i, slf] + cv[i, slf]
                zs = gd[i, sls] + gs[i, sls] + cv[i, sls]
                mv[i, slf] = _sigmoid(zf) * _softplus(zs)
        pltpu.sync_copy(mv, acc.at[dsti.at[pl.ds(c * CH, CH)]], add=True)

    issue(0, 0)
    issue(1, 1)

    def pair(p, _):
        c0 = 2 * p
        wait(c0, 0)
        issue(c0 + 2, 0)
        compute_scatter(c0, 0)
        wait(c0 + 1, 1)

        @pl.when(c0 + 3 < NCHUNK)
        def _():
            issue(c0 + 3, 1)
        compute_scatter(c0 + 1, 1)
        return 0
    lax.fori_loop(0, NCHUNK // 2, pair, 0)
    if NCHUNK % 2:
        c = NCHUNK - 1
        wait(c, 0)
        compute_scatter(c, 0)

    plsc.subcore_barrier()

    pltpu.sync_copy(acc.at[pl.ds(sid * STRIPE, STRIPE)],
                    out_hbm.at[cid, pl.ds(sid * STRIPE, STRIPE)])


_SC_PARAMS = pltpu.CompilerParams(use_tc_tiling_on_sc=False)

_sc_edge = pl.kernel(
    _sc_edge_body,
    out_type=jax.ShapeDtypeStruct((NCORES, NP, D), F32),
    compiler_params=_SC_PARAMS,
    mesh=plsc.VectorSubcoreMesh(core_axis_name="c", subcore_axis_name="s"),
    scratch_types=[
        pltpu.VMEM((EW,), jnp.int32),
        pltpu.VMEM((EW,), jnp.int32),
        pltpu.VMEM((CH, 2 * D), F32),
        pltpu.VMEM((CH, 2 * D), F32),
        pltpu.VMEM((CH, 2 * D), F32),
        pltpu.VMEM((CH, 2 * D), F32),
        pltpu.VMEM((CH, 2 * D), F32),
        pltpu.VMEM((CH, 2 * D), F32),
        pltpu.VMEM((CH, D), F32),
        pltpu.VMEM_SHARED((NP, D), F32),
        pltpu.SemaphoreType.DMA,
        pltpu.SemaphoreType.DMA,
        pltpu.SemaphoreType.DMA,
        pltpu.SemaphoreType.DMA,
        pltpu.SemaphoreType.DMA,
        pltpu.SemaphoreType.DMA,
    ],
)


def _sc_cnt_body(dst_hbm, zero_hbm, out_hbm, dsti, ob, cacc):
    cid = lax.axis_index("c")
    sid = lax.axis_index("s")
    wid = cid * NSUB + sid

    for r in range(STRIPE // CH):
        pltpu.sync_copy(zero_hbm, cacc.at[pl.ds(sid * STRIPE + r * CH, CH)])
    _fill(ob, CH, 16, 1.0)
    plsc.subcore_barrier()

    def chunk(c, _):
        base = wid * EW + c * CH
        pltpu.sync_copy(dst_hbm.at[pl.ds(base, CH)], dsti)
        pltpu.sync_copy(ob, cacc.at[dsti], add=True)
        return 0
    lax.fori_loop(0, NCHUNK, chunk, 0)

    plsc.subcore_barrier()

    pltpu.sync_copy(cacc.at[pl.ds(sid * STRIPE, STRIPE)],
                    out_hbm.at[cid, pl.ds(sid * STRIPE, STRIPE)])


_sc_cnt = pl.kernel(
    _sc_cnt_body,
    out_type=jax.ShapeDtypeStruct((NCORES, NP, 16), F32),
    compiler_params=_SC_PARAMS,
    mesh=plsc.VectorSubcoreMesh(core_axis_name="c", subcore_axis_name="s"),
    scratch_types=[
        pltpu.VMEM((CH,), jnp.int32),
        pltpu.VMEM((CH, 16), F32),
        pltpu.VMEM_SHARED((NP, 16), F32),
    ],
)


def kernel(x, edge_index, edge_attr, batch,
           W_pre, b_pre,
           Wf0, bf0, Ws0, bs0, g0, be0,
           Wf1, bf1, Ws1, bs1, g1, be1,
           Wf2, bf2, Ws2, bs2, g2, be2,
           W_post, b_post, W_out, b_out):
    src = edge_index[0]
    dst = edge_index[1]
    layers = ((Wf0, bf0, Ws0, bs0, g0, be0),
              (Wf1, bf1, Ws1, bs1, g1, be1),
              (Wf2, bf2, Ws2, bs2, g2, be2))

    h = _pre(x, W_pre, b_pre)

    wcat = jnp.stack([
        jnp.concatenate([Wf[2 * D:], Ws[2 * D:]], axis=1)
        for (Wf, _, Ws, _, _, _) in layers])
    bcat = jnp.stack([
        jnp.concatenate([bf, bs]).reshape(1, 2 * D)
        for (_, bf, _, bs, _, _) in layers])
    cfs = _edgeconst(edge_attr, wcat, bcat)

    zero_d = jnp.zeros((CH, D), F32)
    zero_16 = jnp.zeros((CH, 16), F32)
    cnt = _sc_cnt(dst, zero_16)

    acc = None
    for l, (Wf, bf, Ws, bs, g, be) in enumerate(layers):
        wd = jnp.concatenate([Wf[:D], Ws[:D]], axis=1)
        wsrc = jnp.concatenate([Wf[D:2 * D], Ws[D:2 * D]], axis=1)
        if l == 0:
            td, ts = _tables(h, wd, wsrc)
        else:
            h, td, ts = _bn_tables(h, acc, cnt, layers[l - 1][4],
                                   layers[l - 1][5], wd, wsrc)
        acc = _sc_edge(dst, src, td, ts, cfs[l], zero_d)

    return _final(h, acc, cnt, g2, be2, batch,
                  W_post, b_post, W_out, b_out)
